# baseline (speedup 1.0000x reference)
import dataclasses
import functools

import jax
import jax.numpy as jnp
from jax import lax
from jax.experimental import pallas as pl
from jax.experimental.pallas import tpu as pltpu
from jax.experimental.pallas import tpu_sc as plsc

NC = 2
NS = 16
NW = NC * NS
LANES = 16
BE = 128


def _pack_bf16_pair(a, b):
    ua = lax.bitcast_convert_type(a, jnp.uint32)
    ub = lax.bitcast_convert_type(b, jnp.uint32)
    ra = (ua + 0x7FFF + ((ua >> 16) & 1)) >> 16
    rb = (ub + 0x7FFF + ((ub >> 16) & 1)) & jnp.uint32(0xFFFF0000)
    return lax.bitcast_convert_type(ra | rb, jnp.int32)


def _tables_body(h_ref, w_ref, p_ref, q_ref):
    hh = h_ref[...]
    w = w_ref[...]
    s = jnp.sin(hh)
    c = jnp.cos(hh)
    p_ref[...] = _pack_bf16_pair(s * w, c * w)
    q_ref[...] = _pack_bf16_pair(c, -s)


def _build_tables(h, W):
    n, d = h.shape
    out_shape = [
        jax.ShapeDtypeStruct((n, d), jnp.int32),
        jax.ShapeDtypeStruct((n, d), jnp.int32),
    ]
    return pl.pallas_call(_tables_body, out_shape=out_shape)(h, W)


def _sc_gather_dot(p32, q32, ei, bvec):
    e_total = ei.shape[1]
    nchunk = e_total // BE
    nk = -(-nchunk // NW)
    assert (nk - 1) % 2 == 0, "pipeline pair-loop assumes an even body count"
    d = p32.shape[1]
    mesh = plsc.VectorSubcoreMesh(
        core_axis_name="c", subcore_axis_name="s", num_cores=NC, num_subcores=NS
    )
    cp = pltpu.CompilerParams()
    if "needs_layout_passes" in pltpu.CompilerParams.__dataclass_fields__:
        cp = dataclasses.replace(cp, needs_layout_passes=False)

    @functools.partial(
        pl.kernel,
        compiler_params=cp,
        out_type=jax.ShapeDtypeStruct((e_total,), jnp.float32),
        mesh=mesh,
        scratch_types=[
            pltpu.VMEM((2, BE), jnp.int32),
            pltpu.VMEM((2, BE), jnp.int32),
            pltpu.VMEM((BE, d), jnp.int32),
            pltpu.VMEM((BE, d), jnp.int32),
            pltpu.VMEM((BE, d), jnp.int32),
            pltpu.VMEM((BE, d), jnp.int32),
            pltpu.VMEM((BE,), jnp.float32),
            pltpu.VMEM((BE,), jnp.float32),
            pltpu.VMEM((LANES,), jnp.float32),
            pltpu.SemaphoreType.DMA,
            pltpu.SemaphoreType.DMA,
            pltpu.SemaphoreType.DMA,
            pltpu.SemaphoreType.DMA,
            pltpu.SemaphoreType.DMA,
            pltpu.SemaphoreType.DMA,
        ],
    )
    def sc_kernel(p_hbm, q_hbm, ei_hbm, bvec_hbm, out_hbm,
                  idxb0, idxb1, sp0, sp1, sq0, sq1, ob0, ob1, bvec,
                  si0, si1, sg0, sg1, so0, so1):
        idxb = (idxb0, idxb1)
        sp = (sp0, sp1)
        sq = (sq0, sq1)
        ob = (ob0, ob1)
        si = (si0, si1)
        sg = (sg0, sg1)
        so = (so0, so1)
        wid = lax.axis_index("s") * NC + lax.axis_index("c")
        pltpu.sync_copy(bvec_hbm, bvec)
        lane_iota = lax.iota(jnp.int32, LANES)
        last_lane = lane_iota == (LANES - 1)

        def base_of(k):
            c = jnp.minimum(wid + k * NW, nchunk - 1)
            return c * BE

        def idx_slice(k):
            return ei_hbm.at[:, pl.ds(base_of(k), BE)]

        def issue_idx(k, buf, sem):
            pltpu.async_copy(idx_slice(k), idxb[buf], si[sem])

        def wait_idx(k, buf, sem):
            pltpu.make_async_copy(idx_slice(k), idxb[buf], si[sem]).wait()

        def issue_gather(buf):
            pltpu.async_copy(p_hbm.at[idxb[buf].at[0]], sp[buf], sg[buf])
            pltpu.async_copy(q_hbm.at[idxb[buf].at[1]], sq[buf], sg[buf])

        def wait_gather(buf):
            pltpu.make_async_copy(p_hbm.at[idxb[buf].at[0]], sp[buf], sg[buf]).wait()
            pltpu.make_async_copy(q_hbm.at[idxb[buf].at[1]], sq[buf], sg[buf]).wait()

        def out_slice(k):
            return out_hbm.at[pl.ds(base_of(k), BE)]

        def wait_out(k, buf):
            pltpu.make_async_copy(ob[buf], out_slice(k), so[buf]).wait()

        def compute(k, buf):
            srows = sp[buf]
            orows = sq[buf]
            outbuf = ob[buf]

            for t in range(BE // LANES):
                outbuf[pl.ds(t * LANES, LANES)] = jnp.zeros((LANES,), jnp.float32)

            @pl.loop(0, BE, unroll=4)
            def _edge(e):
                accs = [bvec[...], None, None, None]
                for j in range(d // LANES):
                    s32 = srows[e, pl.ds(j * LANES, LANES)]
                    o32 = orows[e, pl.ds(j * LANES, LANES)]
                    sb = plsc.bitcast(s32, jnp.bfloat16)
                    obf = plsc.bitcast(o32, jnp.bfloat16)
                    prod = sb * obf
                    p_lo, p_hi = plsc.unpack(prod, format=plsc.PackFormat.INTERLEAVED)
                    lane = j % 4
                    part = p_lo + p_hi
                    accs[lane] = part if accs[lane] is None else accs[lane] + part
                total = (accs[0] + accs[1]) + (accs[2] + accs[3])
                eidx = jnp.full((LANES,), e, jnp.int32)
                plsc.addupdate_scatter(outbuf, [eidx], total)

            pltpu.async_copy(outbuf, out_slice(k), so[buf])

        pltpu.sync_copy(idx_slice(0), idxb[0])
        issue_gather(0)
        issue_idx(1, 1, 1)

        @pl.loop(0, (nk - 1) // 2)
        def _pair(t):
            for par in (0, 1):
                k = t * 2 + par
                cur, nxt = par, 1 - par
                wait_idx(k + 1, nxt, nxt)
                issue_gather(nxt)
                wait_gather(cur)
                issue_idx(k + 2, cur, cur)

                @pl.when(t >= 1)
                def _():
                    wait_out(k - 2, cur)

                compute(k, cur)

        k_last = nk - 1
        wait_gather(0)
        wait_out(k_last - 2, 0)
        compute(k_last, 0)
        wait_idx(k_last + 1, 1, 1)
        wait_out(k_last - 1, 1)
        wait_out(k_last, 0)

    return sc_kernel(p32, q32, ei, bvec)


def kernel(h, edge_index, W, b):
    e_total = edge_index.shape[1]
    ei = edge_index.astype(jnp.int32)
    p32, q32 = _build_tables(h, W)
    bvec = jnp.zeros((LANES,), jnp.float32).at[0].set(b[0])
    out = _sc_gather_dot(p32, q32, ei, bvec)
    return out.reshape(e_total, 1)


# speedup vs baseline: 1.3266x; 1.3266x over previous
import dataclasses
import functools

import jax
import jax.numpy as jnp
from jax import lax
from jax.experimental import pallas as pl
from jax.experimental.pallas import tpu as pltpu
from jax.experimental.pallas import tpu_sc as plsc

NC = 2
NS = 16
NW = NC * NS
LANES = 16
BE = 128


def _pack_bf16_pair(a, b):
    ua = lax.bitcast_convert_type(a, jnp.uint32)
    ub = lax.bitcast_convert_type(b, jnp.uint32)
    ra = (ua + 0x7FFF + ((ua >> 16) & 1)) >> 16
    rb = (ub + 0x7FFF + ((ub >> 16) & 1)) & jnp.uint32(0xFFFF0000)
    return lax.bitcast_convert_type(ra | rb, jnp.int32)


def _tables_body(h_ref, w_ref, p_ref, q_ref):
    hh = h_ref[...]
    w = w_ref[...]
    s = jnp.sin(hh)
    c = jnp.cos(hh)
    p_ref[...] = _pack_bf16_pair(s * w, c * w)
    q_ref[...] = _pack_bf16_pair(c, -s)


def _build_tables(h, W):
    n, d = h.shape
    out_shape = [
        jax.ShapeDtypeStruct((n, d), jnp.int32),
        jax.ShapeDtypeStruct((n, d), jnp.int32),
    ]
    return pl.pallas_call(_tables_body, out_shape=out_shape)(h, W)


def _sc_gather_dot(p32, q32, ei, bvec):
    e_total = ei.shape[1]
    nchunk = e_total // BE
    nk = -(-nchunk // NW)
    assert (nk - 1) % 2 == 0, "pipeline pair-loop assumes an even body count"
    d = p32.shape[1]
    mesh = plsc.VectorSubcoreMesh(
        core_axis_name="c", subcore_axis_name="s", num_cores=NC, num_subcores=NS
    )
    cp = pltpu.CompilerParams()
    if "needs_layout_passes" in pltpu.CompilerParams.__dataclass_fields__:
        cp = dataclasses.replace(cp, needs_layout_passes=False)

    @functools.partial(
        pl.kernel,
        compiler_params=cp,
        out_type=jax.ShapeDtypeStruct((e_total,), jnp.float32),
        mesh=mesh,
        scratch_types=[
            pltpu.VMEM((2, BE), jnp.int32),
            pltpu.VMEM((2, BE), jnp.int32),
            pltpu.VMEM((BE, d), jnp.int32),
            pltpu.VMEM((BE, d), jnp.int32),
            pltpu.VMEM((BE, d), jnp.int32),
            pltpu.VMEM((BE, d), jnp.int32),
            pltpu.VMEM((BE,), jnp.float32),
            pltpu.VMEM((BE,), jnp.float32),
            pltpu.VMEM((LANES,), jnp.float32),
            pltpu.SemaphoreType.DMA,
            pltpu.SemaphoreType.DMA,
            pltpu.SemaphoreType.DMA,
            pltpu.SemaphoreType.DMA,
            pltpu.SemaphoreType.DMA,
            pltpu.SemaphoreType.DMA,
        ],
    )
    def sc_kernel(p_hbm, q_hbm, ei_hbm, bvec_hbm, out_hbm,
                  idxb0, idxb1, sp0, sp1, sq0, sq1, ob0, ob1, bvec,
                  si0, si1, sg0, sg1, so0, so1):
        idxb = (idxb0, idxb1)
        sp = (sp0, sp1)
        sq = (sq0, sq1)
        ob = (ob0, ob1)
        si = (si0, si1)
        sg = (sg0, sg1)
        so = (so0, so1)
        wid = lax.axis_index("s") * NC + lax.axis_index("c")
        pltpu.sync_copy(bvec_hbm, bvec)
        lane_iota = lax.iota(jnp.int32, LANES)
        last_lane = lane_iota == (LANES - 1)

        def base_of(k):
            c = jnp.minimum(wid + k * NW, nchunk - 1)
            return c * BE

        def idx_slice(k):
            return ei_hbm.at[:, pl.ds(base_of(k), BE)]

        def issue_idx(k, buf, sem):
            pltpu.async_copy(idx_slice(k), idxb[buf], si[sem])

        def wait_idx(k, buf, sem):
            pltpu.make_async_copy(idx_slice(k), idxb[buf], si[sem]).wait()

        def issue_gather(buf):
            pltpu.async_copy(p_hbm.at[idxb[buf].at[0]], sp[buf], sg[buf])
            pltpu.async_copy(q_hbm.at[idxb[buf].at[1]], sq[buf], sg[buf])

        def wait_gather(buf):
            pltpu.make_async_copy(p_hbm.at[idxb[buf].at[0]], sp[buf], sg[buf]).wait()
            pltpu.make_async_copy(q_hbm.at[idxb[buf].at[1]], sq[buf], sg[buf]).wait()

        def out_slice(k):
            return out_hbm.at[pl.ds(base_of(k), BE)]

        def wait_out(k, buf):
            pltpu.make_async_copy(ob[buf], out_slice(k), so[buf]).wait()

        def compute(k, buf):
            srows = sp[buf]
            orows = sq[buf]
            outbuf = ob[buf]

            @pl.loop(0, BE, unroll=4)
            def _edge(e):
                accs = [bvec[...], None, None, None]
                for j in range(d // LANES):
                    s32 = srows[e, pl.ds(j * LANES, LANES)]
                    o32 = orows[e, pl.ds(j * LANES, LANES)]
                    sb = plsc.bitcast(s32, jnp.bfloat16)
                    obf = plsc.bitcast(o32, jnp.bfloat16)
                    prod = sb * obf
                    p_lo, p_hi = plsc.unpack(prod, format=plsc.PackFormat.INTERLEAVED)
                    lane = j % 4
                    part = p_lo + p_hi
                    accs[lane] = part if accs[lane] is None else accs[lane] + part
                total = plsc.cumsum((accs[0] + accs[1]) + (accs[2] + accs[3]))
                eidx = jnp.full((LANES,), e, jnp.int32)
                plsc.store_scatter(outbuf, [eidx], total, mask=last_lane)

            pltpu.async_copy(outbuf, out_slice(k), so[buf])

        pltpu.sync_copy(idx_slice(0), idxb[0])
        issue_gather(0)
        issue_idx(1, 1, 1)

        @pl.loop(0, (nk - 1) // 2)
        def _pair(t):
            for par in (0, 1):
                k = t * 2 + par
                cur, nxt = par, 1 - par
                wait_idx(k + 1, nxt, nxt)
                issue_gather(nxt)
                wait_gather(cur)
                issue_idx(k + 2, cur, cur)

                @pl.when(t >= 1)
                def _():
                    wait_out(k - 2, cur)

                compute(k, cur)

        k_last = nk - 1
        wait_gather(0)
        wait_out(k_last - 2, 0)
        compute(k_last, 0)
        wait_idx(k_last + 1, 1, 1)
        wait_out(k_last - 1, 1)
        wait_out(k_last, 0)

    return sc_kernel(p32, q32, ei, bvec)


def kernel(h, edge_index, W, b):
    e_total = edge_index.shape[1]
    ei = edge_index.astype(jnp.int32)
    p32, q32 = _build_tables(h, W)
    bvec = jnp.zeros((LANES,), jnp.float32).at[0].set(b[0])
    out = _sc_gather_dot(p32, q32, ei, bvec)
    return out.reshape(e_total, 1)


# speedup vs baseline: 2.2157x; 1.6702x over previous
import dataclasses
import functools

import jax
import jax.numpy as jnp
from jax import lax
from jax.experimental import pallas as pl
from jax.experimental.pallas import tpu as pltpu
from jax.experimental.pallas import tpu_sc as plsc

NC = 2
NS = 16
NW = NC * NS
LANES = 16
BE = 128


def _pack_bf16_pair(a, b):
    ua = lax.bitcast_convert_type(a, jnp.uint32)
    ub = lax.bitcast_convert_type(b, jnp.uint32)
    ra = (ua + 0x7FFF + ((ua >> 16) & 1)) >> 16
    rb = (ub + 0x7FFF + ((ub >> 16) & 1)) & jnp.uint32(0xFFFF0000)
    return lax.bitcast_convert_type(ra | rb, jnp.int32)


def _tables_body(h_ref, w_ref, p_ref, q_ref):
    hh = h_ref[...]
    w = w_ref[...]
    s = jnp.sin(hh)
    c = jnp.cos(hh)
    p_ref[...] = _pack_bf16_pair(s * w, c * w)
    q_ref[...] = _pack_bf16_pair(c, -s)


def _build_tables(h, W):
    n, d = h.shape
    out_shape = [
        jax.ShapeDtypeStruct((n, d), jnp.int32),
        jax.ShapeDtypeStruct((n, d), jnp.int32),
    ]
    return pl.pallas_call(_tables_body, out_shape=out_shape)(h, W)


def _sc_gather_dot(p32, q32, ei, bvec):
    e_total = ei.shape[1]
    nchunk = e_total // BE
    nk = -(-nchunk // NW)
    assert (nk - 1) % 2 == 0, "pipeline pair-loop assumes an even body count"
    d = p32.shape[1]
    mesh = plsc.VectorSubcoreMesh(
        core_axis_name="c", subcore_axis_name="s", num_cores=NC, num_subcores=NS
    )
    cp = pltpu.CompilerParams()
    if "needs_layout_passes" in pltpu.CompilerParams.__dataclass_fields__:
        cp = dataclasses.replace(cp, needs_layout_passes=False)

    @functools.partial(
        pl.kernel,
        compiler_params=cp,
        out_type=jax.ShapeDtypeStruct((e_total,), jnp.float32),
        mesh=mesh,
        scratch_types=[
            pltpu.VMEM((2, BE), jnp.int32),
            pltpu.VMEM((2, BE), jnp.int32),
            pltpu.VMEM((BE, d), jnp.int32),
            pltpu.VMEM((BE, d), jnp.int32),
            pltpu.VMEM((BE, d), jnp.int32),
            pltpu.VMEM((BE, d), jnp.int32),
            pltpu.VMEM((BE,), jnp.float32),
            pltpu.VMEM((BE,), jnp.float32),
            pltpu.VMEM((LANES,), jnp.float32),
            pltpu.SemaphoreType.DMA,
            pltpu.SemaphoreType.DMA,
            pltpu.SemaphoreType.DMA,
            pltpu.SemaphoreType.DMA,
            pltpu.SemaphoreType.DMA,
            pltpu.SemaphoreType.DMA,
        ],
    )
    def sc_kernel(p_hbm, q_hbm, ei_hbm, bvec_hbm, out_hbm,
                  idxb0, idxb1, sp0, sp1, sq0, sq1, ob0, ob1, bvec,
                  si0, si1, sg0, sg1, so0, so1):
        idxb = (idxb0, idxb1)
        sp = (sp0, sp1)
        sq = (sq0, sq1)
        ob = (ob0, ob1)
        si = (si0, si1)
        sg = (sg0, sg1)
        so = (so0, so1)
        wid = lax.axis_index("s") * NC + lax.axis_index("c")
        pltpu.sync_copy(bvec_hbm, bvec)
        lane_iota = lax.iota(jnp.int32, LANES)
        last_lane = lane_iota == (LANES - 1)

        def base_of(k):
            c = jnp.minimum(wid + k * NW, nchunk - 1)
            return c * BE

        def idx_slice(k):
            return ei_hbm.at[:, pl.ds(base_of(k), BE)]

        def issue_idx(k, buf, sem):
            pltpu.async_copy(idx_slice(k), idxb[buf], si[sem])

        def wait_idx(k, buf, sem):
            pltpu.make_async_copy(idx_slice(k), idxb[buf], si[sem]).wait()

        def issue_gather(buf):
            pltpu.async_copy(p_hbm.at[idxb[buf].at[0]], sp[buf], sg[buf])
            pltpu.async_copy(q_hbm.at[idxb[buf].at[1]], sq[buf], sg[buf])

        def wait_gather(buf):
            pltpu.make_async_copy(p_hbm.at[idxb[buf].at[0]], sp[buf], sg[buf]).wait()
            pltpu.make_async_copy(q_hbm.at[idxb[buf].at[1]], sq[buf], sg[buf]).wait()

        def out_slice(k):
            return out_hbm.at[pl.ds(base_of(k), BE)]

        def wait_out(k, buf):
            pltpu.make_async_copy(ob[buf], out_slice(k), so[buf]).wait()

        def compute(k, buf):
            srows = sp[buf]
            orows = sq[buf]
            outbuf = ob[buf]

            @plsc.parallel_loop(0, BE, unroll=2)
            def _edge(e):
                accs = [bvec[...], None, None, None]
                for j in range(d // LANES):
                    s32 = srows[e, pl.ds(j * LANES, LANES)]
                    o32 = orows[e, pl.ds(j * LANES, LANES)]
                    sb = plsc.bitcast(s32, jnp.bfloat16)
                    obf = plsc.bitcast(o32, jnp.bfloat16)
                    prod = sb * obf
                    p_lo, p_hi = plsc.unpack(prod, format=plsc.PackFormat.INTERLEAVED)
                    lane = j % 4
                    part = p_lo + p_hi
                    accs[lane] = part if accs[lane] is None else accs[lane] + part
                total = plsc.cumsum((accs[0] + accs[1]) + (accs[2] + accs[3]))
                eidx = jnp.full((LANES,), e, jnp.int32)
                plsc.store_scatter(outbuf, [eidx], total, mask=last_lane)

            pltpu.async_copy(outbuf, out_slice(k), so[buf])

        pltpu.sync_copy(idx_slice(0), idxb[0])
        issue_gather(0)
        issue_idx(1, 1, 1)

        @pl.loop(0, (nk - 1) // 2)
        def _pair(t):
            for par in (0, 1):
                k = t * 2 + par
                cur, nxt = par, 1 - par
                wait_idx(k + 1, nxt, nxt)
                issue_gather(nxt)
                wait_gather(cur)
                issue_idx(k + 2, cur, cur)

                @pl.when(t >= 1)
                def _():
                    wait_out(k - 2, cur)

                compute(k, cur)

        k_last = nk - 1
        wait_gather(0)
        wait_out(k_last - 2, 0)
        compute(k_last, 0)
        wait_idx(k_last + 1, 1, 1)
        wait_out(k_last - 1, 1)
        wait_out(k_last, 0)

    return sc_kernel(p32, q32, ei, bvec)


def kernel(h, edge_index, W, b):
    e_total = edge_index.shape[1]
    ei = edge_index.astype(jnp.int32)
    p32, q32 = _build_tables(h, W)
    bvec = jnp.zeros((LANES,), jnp.float32).at[0].set(b[0])
    out = _sc_gather_dot(p32, q32, ei, bvec)
    return out.reshape(e_total, 1)


# speedup vs baseline: 2.4737x; 1.1165x over previous
import dataclasses
import functools

import jax
import jax.numpy as jnp
from jax import lax
from jax.experimental import pallas as pl
from jax.experimental.pallas import tpu as pltpu
from jax.experimental.pallas import tpu_sc as plsc

NC = 2
NS = 16
NW = NC * NS
LANES = 16
BE = 128


def _pack_bf16_pair(a, b):
    ua = lax.bitcast_convert_type(a, jnp.uint32)
    ub = lax.bitcast_convert_type(b, jnp.uint32)
    ra = (ua + 0x7FFF + ((ua >> 16) & 1)) >> 16
    rb = (ub + 0x7FFF + ((ub >> 16) & 1)) & jnp.uint32(0xFFFF0000)
    return lax.bitcast_convert_type(ra | rb, jnp.int32)


def _tables_body(h_ref, w_ref, p_ref, q_ref):
    hh = h_ref[...]
    w = w_ref[...]
    s = jnp.sin(hh)
    c = jnp.cos(hh)
    p_ref[...] = _pack_bf16_pair(s * w, c * w)
    q_ref[...] = _pack_bf16_pair(c, -s)


def _build_tables(h, W):
    n, d = h.shape
    out_shape = [
        jax.ShapeDtypeStruct((n, d), jnp.int32),
        jax.ShapeDtypeStruct((n, d), jnp.int32),
    ]
    return pl.pallas_call(_tables_body, out_shape=out_shape)(h, W)


def _sc_gather_dot(p32, q32, ei, bvec):
    e_total = ei.shape[1]
    nchunk = e_total // BE
    nk = -(-nchunk // NW)
    assert nk >= 5, "pipeline prologue/epilogue assume at least 5 chunks"
    d = p32.shape[1]
    mesh = plsc.VectorSubcoreMesh(
        core_axis_name="c", subcore_axis_name="s", num_cores=NC, num_subcores=NS
    )
    cp = pltpu.CompilerParams()
    if "needs_layout_passes" in pltpu.CompilerParams.__dataclass_fields__:
        cp = dataclasses.replace(cp, needs_layout_passes=False)

    @functools.partial(
        pl.kernel,
        compiler_params=cp,
        out_type=jax.ShapeDtypeStruct((e_total,), jnp.float32),
        mesh=mesh,
        scratch_types=(
            [pltpu.VMEM((2, BE), jnp.int32) for _ in range(3)]
            + [pltpu.VMEM((BE, d), jnp.int32) for _ in range(3)]
            + [pltpu.VMEM((BE, d), jnp.int32) for _ in range(3)]
            + [pltpu.VMEM((BE,), jnp.float32) for _ in range(3)]
            + [pltpu.VMEM((LANES,), jnp.float32)]
            + [pltpu.SemaphoreType.DMA for _ in range(9)]
        ),
    )
    def sc_kernel(p_hbm, q_hbm, ei_hbm, bvec_hbm, out_hbm,
                  idxb0, idxb1, idxb2, sp0, sp1, sp2, sq0, sq1, sq2,
                  ob0, ob1, ob2, bvec,
                  si0, si1, si2, sg0, sg1, sg2, so0, so1, so2):
        idxb = (idxb0, idxb1, idxb2)
        sp = (sp0, sp1, sp2)
        sq = (sq0, sq1, sq2)
        ob = (ob0, ob1, ob2)
        si = (si0, si1, si2)
        sg = (sg0, sg1, sg2)
        so = (so0, so1, so2)
        wid = lax.axis_index("s") * NC + lax.axis_index("c")
        pltpu.sync_copy(bvec_hbm, bvec)
        lane_iota = lax.iota(jnp.int32, LANES)
        last_lane = lane_iota == (LANES - 1)

        def base_of(k):
            c = jnp.minimum(wid + k * NW, nchunk - 1)
            return c * BE

        def idx_slice(k):
            return ei_hbm.at[:, pl.ds(base_of(k), BE)]

        def issue_idx(k, buf, sem):
            pltpu.async_copy(idx_slice(k), idxb[buf], si[sem])

        def wait_idx(k, buf, sem):
            pltpu.make_async_copy(idx_slice(k), idxb[buf], si[sem]).wait()

        def issue_gather(buf):
            pltpu.async_copy(p_hbm.at[idxb[buf].at[0]], sp[buf], sg[buf])
            pltpu.async_copy(q_hbm.at[idxb[buf].at[1]], sq[buf], sg[buf])

        def wait_gather(buf):
            pltpu.make_async_copy(p_hbm.at[idxb[buf].at[0]], sp[buf], sg[buf]).wait()
            pltpu.make_async_copy(q_hbm.at[idxb[buf].at[1]], sq[buf], sg[buf]).wait()

        def out_slice(k):
            return out_hbm.at[pl.ds(base_of(k), BE)]

        def wait_out(k, buf):
            pltpu.make_async_copy(ob[buf], out_slice(k), so[buf]).wait()

        def compute(k, buf):
            srows = sp[buf]
            orows = sq[buf]
            outbuf = ob[buf]

            @plsc.parallel_loop(0, BE, unroll=2)
            def _edge(e):
                accs = [bvec[...], None, None, None]
                for j in range(d // LANES):
                    s32 = srows[e, pl.ds(j * LANES, LANES)]
                    o32 = orows[e, pl.ds(j * LANES, LANES)]
                    sb = plsc.bitcast(s32, jnp.bfloat16)
                    obf = plsc.bitcast(o32, jnp.bfloat16)
                    prod = sb * obf
                    p_lo, p_hi = plsc.unpack(prod, format=plsc.PackFormat.INTERLEAVED)
                    lane = j % 4
                    part = p_lo + p_hi
                    accs[lane] = part if accs[lane] is None else accs[lane] + part
                total = plsc.cumsum((accs[0] + accs[1]) + (accs[2] + accs[3]))
                eidx = jnp.full((LANES,), e, jnp.int32)
                plsc.store_scatter(outbuf, [eidx], total, mask=last_lane)

            pltpu.async_copy(outbuf, out_slice(k), so[buf])

        def body(k, par):
            kp2 = (par + 2) % 3
            wait_idx(k + 2, kp2, kp2)
            issue_gather(kp2)
            wait_gather(par)
            issue_idx(k + 3, par, par)

            @pl.when(k >= 3)
            def _():
                wait_out(k - 3, par)

            compute(k, par)

        pltpu.sync_copy(idx_slice(0), idxb[0])
        issue_gather(0)
        issue_idx(1, 1, 1)
        issue_idx(2, 2, 2)
        wait_idx(1, 1, 1)
        issue_gather(1)

        @pl.loop(0, (nk - 2) // 3)
        def _trip(t):
            for par in range(3):
                body(3 * t + par, par)

        for k in range(nk - 2 - (nk - 2) % 3, nk - 2):
            body(k, k % 3)

        for k in (nk - 2, nk - 1):
            wait_gather(k % 3)
            wait_out(k - 3, k % 3)
            compute(k, k % 3)
        wait_idx(nk, (nk - 3) % 3, (nk - 3) % 3)
        for k in (nk - 2, nk - 1, nk - 3):
            wait_out(k, k % 3)

    return sc_kernel(p32, q32, ei, bvec)


def kernel(h, edge_index, W, b):
    e_total = edge_index.shape[1]
    ei = edge_index.astype(jnp.int32)
    p32, q32 = _build_tables(h, W)
    bvec = jnp.zeros((LANES,), jnp.float32).at[0].set(b[0])
    out = _sc_gather_dot(p32, q32, ei, bvec)
    return out.reshape(e_total, 1)
